# baseline (device time: 236815 ns/iter reference)
import jax
import jax.numpy as jnp
from jax import lax
from jax.experimental import pallas as pl
from jax.experimental.pallas import tpu as pltpu

N_ROWS = 8192
N_COLS = 2048
HALF = N_ROWS // 2
SIZES = [32, 32, 64, 128, 256] + [512] * 6 + [256, 128, 64, 32, 32]
assert sum(SIZES) == HALF
OFFS = [sum(SIZES[:i]) for i in range(len(SIZES))]
N_CHUNKS = len(SIZES)
MAXC = max(SIZES)
EPS = 1e-6


def kernel(partial, resid, gamma):
    gamma2 = gamma.reshape(1, N_COLS)

    def body(
        partial_ref,
        resid_ref,
        gamma_ref,
        out_ref,
        own_f32,
        resid_v,
        own_bf16,
        recv_y,
        out_v,
        out_bf16,
        recv_x,
        pload_sems,
        rload_sems,
        store_sems,
        xstore_sems,
        ysend_sems,
        yrecv_sems,
        xsend_sems,
        xrecv_sems,
    ):
        my_x = lax.axis_index("x")
        my_y = lax.axis_index("y")
        ynbr = (my_x, 1 - my_y)
        xnbr = (1 - my_x, my_y)
        base_own = my_x * HALF
        base_oth = (1 - my_x) * HALF

        barrier_sem = pltpu.get_barrier_semaphore()
        for nbr in (ynbr, xnbr):
            pl.semaphore_signal(
                barrier_sem, inc=1, device_id=nbr,
                device_id_type=pl.DeviceIdType.MESH,
            )
        pl.semaphore_wait(barrier_sem, 2)

        def start_loads(i):
            s = i % 2
            sz = SIZES[i]
            rows = pl.ds(base_own + OFFS[i], sz)
            ld_p = pltpu.make_async_copy(
                partial_ref.at[0, rows, :],
                own_f32.at[s, pl.ds(0, sz), :],
                pload_sems.at[s],
            )
            ld_r = pltpu.make_async_copy(
                resid_ref.at[rows, :],
                resid_v.at[s, pl.ds(0, sz), :],
                rload_sems.at[s],
            )
            ld_p.start()
            ld_r.start()
            return ld_p, ld_r

        def y_rdma(i):
            sz = SIZES[i]
            return pltpu.make_async_remote_copy(
                src_ref=own_bf16.at[i % 2, pl.ds(0, sz), :],
                dst_ref=recv_y.at[i % 4, pl.ds(0, sz), :],
                send_sem=ysend_sems.at[i % 2],
                recv_sem=yrecv_sems.at[i % 4],
                device_id=ynbr,
                device_id_type=pl.DeviceIdType.MESH,
            )

        def x_rdma(i):
            sz = SIZES[i]
            return pltpu.make_async_remote_copy(
                src_ref=out_bf16.at[i % 2, pl.ds(0, sz), :],
                dst_ref=recv_x.at[i % 4, pl.ds(0, sz), :],
                send_sem=xsend_sems.at[i % 2],
                recv_sem=xrecv_sems.at[i % 4],
                device_id=xnbr,
                device_id_type=pl.DeviceIdType.MESH,
            )

        loads = [None] * N_CHUNKS
        ydescs = [None] * N_CHUNKS
        xdescs = [None] * N_CHUNKS
        stores = [None] * N_CHUNKS
        xstores = [None] * N_CHUNKS

        def handle_xrecv(j):
            xdescs[j].wait_recv()
            st = pltpu.make_async_copy(
                recv_x.at[j % 4, pl.ds(0, SIZES[j]), :],
                out_ref.at[pl.ds(base_oth + OFFS[j], SIZES[j]), :],
                xstore_sems.at[j % 2],
            )
            st.start()
            xstores[j] = st

        loads[0] = start_loads(0)
        loads[1] = start_loads(1)
        loads[0][0].wait()
        own_bf16[0, : SIZES[0]] = own_f32[0, : SIZES[0]].astype(jnp.bfloat16)
        ydescs[0] = y_rdma(0)
        ydescs[0].start()

        for i in range(N_CHUNKS):
            s = i % 2
            sz = SIZES[i]
            if i + 1 < N_CHUNKS:
                loads[i + 1][0].wait()
                if i >= 1:
                    ydescs[i - 1].wait_send()
                nsz = SIZES[i + 1]
                own_bf16[(i + 1) % 2, :nsz] = own_f32[
                    (i + 1) % 2, :nsz
                ].astype(jnp.bfloat16)
                ydescs[i + 1] = y_rdma(i + 1)
                ydescs[i + 1].start()

            ydescs[i].wait_recv()

            loads[i][1].wait()
            y = (
                own_f32[s, :sz]
                + recv_y[i % 4, :sz].astype(jnp.float32)
                + resid_v[s, :sz]
            )
            ms = jnp.mean(y * y, axis=-1, keepdims=True)
            out_v[:sz] = y * lax.rsqrt(ms + EPS) * gamma_ref[...]

            if i >= 2:
                xdescs[i - 2].wait_send()
                stores[i - 2].wait()
                xstores[i - 2].wait()
            out_bf16[s, :sz] = out_v[:sz].astype(jnp.bfloat16)
            st = pltpu.make_async_copy(
                out_bf16.at[s, pl.ds(0, sz), :],
                out_ref.at[pl.ds(base_own + OFFS[i], sz), :],
                store_sems.at[s],
            )
            st.start()
            stores[i] = st
            xdescs[i] = x_rdma(i)
            xdescs[i].start()

            if i + 2 < N_CHUNKS:
                loads[i + 2] = start_loads(i + 2)

            if i >= 1:
                handle_xrecv(i - 1)

        handle_xrecv(N_CHUNKS - 1)
        for k in (N_CHUNKS - 2, N_CHUNKS - 1):
            stores[k].wait()
            xstores[k].wait()
            ydescs[k].wait_send()
            xdescs[k].wait_send()

    return pl.pallas_call(
        body,
        out_shape=jax.ShapeDtypeStruct((N_ROWS, N_COLS), jnp.bfloat16),
        in_specs=[
            pl.BlockSpec(memory_space=pltpu.MemorySpace.HBM),
            pl.BlockSpec(memory_space=pltpu.MemorySpace.HBM),
            pl.BlockSpec(memory_space=pltpu.VMEM),
        ],
        out_specs=pl.BlockSpec(memory_space=pltpu.MemorySpace.HBM),
        scratch_shapes=[
            pltpu.VMEM((2, MAXC, N_COLS), jnp.float32),
            pltpu.VMEM((2, MAXC, N_COLS), jnp.float32),
            pltpu.VMEM((2, MAXC, N_COLS), jnp.bfloat16),
            pltpu.VMEM((4, MAXC, N_COLS), jnp.bfloat16),
            pltpu.VMEM((MAXC, N_COLS), jnp.float32),
            pltpu.VMEM((2, MAXC, N_COLS), jnp.bfloat16),
            pltpu.VMEM((4, MAXC, N_COLS), jnp.bfloat16),
            pltpu.SemaphoreType.DMA((2,)),
            pltpu.SemaphoreType.DMA((2,)),
            pltpu.SemaphoreType.DMA((2,)),
            pltpu.SemaphoreType.DMA((2,)),
            pltpu.SemaphoreType.DMA((2,)),
            pltpu.SemaphoreType.DMA((4,)),
            pltpu.SemaphoreType.DMA((2,)),
            pltpu.SemaphoreType.DMA((4,)),
        ],
        compiler_params=pltpu.CompilerParams(
            collective_id=0,
            vmem_limit_bytes=60 * 1024 * 1024,
        ),
    )(partial, resid, gamma2)


# device time: 229757 ns/iter; 1.0307x vs baseline; 1.0307x over previous
import jax
import jax.numpy as jnp
from jax import lax
from jax.experimental import pallas as pl
from jax.experimental.pallas import tpu as pltpu

N_ROWS = 8192
N_COLS = 2048
HALF = N_ROWS // 2
CHUNK = 64
N_CHUNKS = HALF // CHUNK
EPS = 1e-6


def kernel(partial, resid, gamma):
    gamma2 = gamma.reshape(1, N_COLS)

    def body(
        partial_ref,
        resid_ref,
        gamma_ref,
        out_ref,
        own_f32,
        resid_v,
        own_bf16,
        recv_y,
        out_v,
        out_bf16,
        recv_x,
        pload_sems,
        rload_sems,
        store_sems,
        xstore_sems,
        ysend_sems,
        yrecv_sems,
        xsend_sems,
        xrecv_sems,
    ):
        my_x = lax.axis_index("x")
        my_y = lax.axis_index("y")
        ynbr = (my_x, 1 - my_y)
        xnbr = (1 - my_x, my_y)
        base_own = my_x * HALF
        base_oth = (1 - my_x) * HALF

        barrier_sem = pltpu.get_barrier_semaphore()
        for nbr in (ynbr, xnbr):
            pl.semaphore_signal(
                barrier_sem, inc=1, device_id=nbr,
                device_id_type=pl.DeviceIdType.MESH,
            )
        pl.semaphore_wait(barrier_sem, 2)

        def start_loads(i):
            s = i % 2
            rows = pl.ds(base_own + i * CHUNK, CHUNK)
            ld_p = pltpu.make_async_copy(
                partial_ref.at[0, rows, :], own_f32.at[s], pload_sems.at[s]
            )
            ld_r = pltpu.make_async_copy(
                resid_ref.at[rows, :], resid_v.at[s], rload_sems.at[s]
            )
            ld_p.start()
            ld_r.start()
            return ld_p, ld_r

        def y_rdma(i):
            return pltpu.make_async_remote_copy(
                src_ref=own_bf16.at[i % 2],
                dst_ref=recv_y.at[i % 4],
                send_sem=ysend_sems.at[i % 2],
                recv_sem=yrecv_sems.at[i % 4],
                device_id=ynbr,
                device_id_type=pl.DeviceIdType.MESH,
            )

        def x_rdma(i):
            return pltpu.make_async_remote_copy(
                src_ref=out_bf16.at[i % 2],
                dst_ref=recv_x.at[i % 4],
                send_sem=xsend_sems.at[i % 2],
                recv_sem=xrecv_sems.at[i % 4],
                device_id=xnbr,
                device_id_type=pl.DeviceIdType.MESH,
            )

        loads = [None] * N_CHUNKS
        ydescs = [None] * N_CHUNKS
        xdescs = [None] * N_CHUNKS
        stores = [None] * N_CHUNKS
        xstores = [None] * N_CHUNKS

        def handle_xrecv(j):
            xdescs[j].wait_recv()
            st = pltpu.make_async_copy(
                recv_x.at[j % 4],
                out_ref.at[pl.ds(base_oth + j * CHUNK, CHUNK), :],
                xstore_sems.at[j % 2],
            )
            st.start()
            xstores[j] = st

        loads[0] = start_loads(0)
        loads[1] = start_loads(1)
        loads[0][0].wait()
        own_bf16[0] = own_f32[0].astype(jnp.bfloat16)
        ydescs[0] = y_rdma(0)
        ydescs[0].start()

        for i in range(N_CHUNKS):
            s = i % 2
            if i + 1 < N_CHUNKS:
                loads[i + 1][0].wait()
                if i >= 1:
                    ydescs[i - 1].wait_send()
                own_bf16[(i + 1) % 2] = own_f32[(i + 1) % 2].astype(jnp.bfloat16)
                ydescs[i + 1] = y_rdma(i + 1)
                ydescs[i + 1].start()

            ydescs[i].wait_recv()

            loads[i][1].wait()
            y = own_f32[s] + recv_y[i % 4].astype(jnp.float32) + resid_v[s]
            ms = jnp.mean(y * y, axis=-1, keepdims=True)
            out_v[...] = y * lax.rsqrt(ms + EPS) * gamma_ref[...]

            if i >= 2:
                xdescs[i - 2].wait_send()
                stores[i - 2].wait()
                xstores[i - 2].wait()
            out_bf16[s] = out_v[...].astype(jnp.bfloat16)
            st = pltpu.make_async_copy(
                out_bf16.at[s],
                out_ref.at[pl.ds(base_own + i * CHUNK, CHUNK), :],
                store_sems.at[s],
            )
            st.start()
            stores[i] = st
            xdescs[i] = x_rdma(i)
            xdescs[i].start()

            if i + 2 < N_CHUNKS:
                loads[i + 2] = start_loads(i + 2)

            if i >= 1:
                handle_xrecv(i - 1)

        handle_xrecv(N_CHUNKS - 1)
        for k in (N_CHUNKS - 2, N_CHUNKS - 1):
            stores[k].wait()
            xstores[k].wait()
            ydescs[k].wait_send()
            xdescs[k].wait_send()

    return pl.pallas_call(
        body,
        out_shape=jax.ShapeDtypeStruct((N_ROWS, N_COLS), jnp.bfloat16),
        in_specs=[
            pl.BlockSpec(memory_space=pltpu.MemorySpace.HBM),
            pl.BlockSpec(memory_space=pltpu.MemorySpace.HBM),
            pl.BlockSpec(memory_space=pltpu.VMEM),
        ],
        out_specs=pl.BlockSpec(memory_space=pltpu.MemorySpace.HBM),
        scratch_shapes=[
            pltpu.VMEM((2, CHUNK, N_COLS), jnp.float32),
            pltpu.VMEM((2, CHUNK, N_COLS), jnp.float32),
            pltpu.VMEM((2, CHUNK, N_COLS), jnp.bfloat16),
            pltpu.VMEM((4, CHUNK, N_COLS), jnp.bfloat16),
            pltpu.VMEM((CHUNK, N_COLS), jnp.float32),
            pltpu.VMEM((2, CHUNK, N_COLS), jnp.bfloat16),
            pltpu.VMEM((4, CHUNK, N_COLS), jnp.bfloat16),
            pltpu.SemaphoreType.DMA((2,)),
            pltpu.SemaphoreType.DMA((2,)),
            pltpu.SemaphoreType.DMA((2,)),
            pltpu.SemaphoreType.DMA((2,)),
            pltpu.SemaphoreType.DMA((2,)),
            pltpu.SemaphoreType.DMA((4,)),
            pltpu.SemaphoreType.DMA((2,)),
            pltpu.SemaphoreType.DMA((4,)),
        ],
        compiler_params=pltpu.CompilerParams(
            collective_id=0,
            vmem_limit_bytes=60 * 1024 * 1024,
        ),
    )(partial, resid, gamma2)


# device time: 219686 ns/iter; 1.0780x vs baseline; 1.0458x over previous
import jax
import jax.numpy as jnp
from jax import lax
from jax.experimental import pallas as pl
from jax.experimental.pallas import tpu as pltpu

N_ROWS = 8192
N_COLS = 2048
HALF = N_ROWS // 2
SIZES = [32, 32, 32, 32] + [128] * 30 + [64, 32, 32]
assert sum(SIZES) == HALF
OFFS = [sum(SIZES[:i]) for i in range(len(SIZES))]
N_CHUNKS = len(SIZES)
MAXC = max(SIZES)
EPS = 1e-6


def kernel(partial, resid, gamma):
    gamma2 = gamma.reshape(1, N_COLS)

    def body(
        partial_ref,
        resid_ref,
        gamma_ref,
        out_ref,
        own_f32,
        resid_v,
        own_bf16,
        recv_y,
        out_v,
        out_bf16,
        recv_x,
        pload_sems,
        rload_sems,
        store_sems,
        xstore_sems,
        ysend_sems,
        yrecv_sems,
        xsend_sems,
        xrecv_sems,
    ):
        my_x = lax.axis_index("x")
        my_y = lax.axis_index("y")
        ynbr = (my_x, 1 - my_y)
        xnbr = (1 - my_x, my_y)
        base_own = my_x * HALF
        base_oth = (1 - my_x) * HALF

        def start_loads(i):
            s = i % 2
            sz = SIZES[i]
            rows = pl.ds(base_own + OFFS[i], sz)
            ld_p = pltpu.make_async_copy(
                partial_ref.at[0, rows, :],
                own_f32.at[s, pl.ds(0, sz), :],
                pload_sems.at[s],
            )
            ld_r = pltpu.make_async_copy(
                resid_ref.at[rows, :],
                resid_v.at[s, pl.ds(0, sz), :],
                rload_sems.at[s],
            )
            ld_p.start()
            ld_r.start()
            return ld_p, ld_r

        def y_rdma(i):
            sz = SIZES[i]
            return pltpu.make_async_remote_copy(
                src_ref=own_bf16.at[i % 2, pl.ds(0, sz), :],
                dst_ref=recv_y.at[i % 4, pl.ds(0, sz), :],
                send_sem=ysend_sems.at[i % 2],
                recv_sem=yrecv_sems.at[i % 4],
                device_id=ynbr,
                device_id_type=pl.DeviceIdType.MESH,
            )

        def x_rdma(i):
            sz = SIZES[i]
            return pltpu.make_async_remote_copy(
                src_ref=out_bf16.at[i % 2, pl.ds(0, sz), :],
                dst_ref=recv_x.at[i % 4, pl.ds(0, sz), :],
                send_sem=xsend_sems.at[i % 2],
                recv_sem=xrecv_sems.at[i % 4],
                device_id=xnbr,
                device_id_type=pl.DeviceIdType.MESH,
            )

        loads = [None] * N_CHUNKS
        ydescs = [None] * N_CHUNKS
        xdescs = [None] * N_CHUNKS
        stores = [None] * N_CHUNKS
        xstores = [None] * N_CHUNKS

        def handle_xrecv(j):
            xdescs[j].wait_recv()
            st = pltpu.make_async_copy(
                recv_x.at[j % 4, pl.ds(0, SIZES[j]), :],
                out_ref.at[pl.ds(base_oth + OFFS[j], SIZES[j]), :],
                xstore_sems.at[j % 2],
            )
            st.start()
            xstores[j] = st

        loads[0] = start_loads(0)
        loads[1] = start_loads(1)

        barrier_sem = pltpu.get_barrier_semaphore()
        for nbr in (ynbr, xnbr):
            pl.semaphore_signal(
                barrier_sem, inc=1, device_id=nbr,
                device_id_type=pl.DeviceIdType.MESH,
            )
        pl.semaphore_wait(barrier_sem, 2)

        loads[0][0].wait()
        own_bf16[0, : SIZES[0]] = own_f32[0, : SIZES[0]].astype(jnp.bfloat16)
        ydescs[0] = y_rdma(0)
        ydescs[0].start()

        for i in range(N_CHUNKS):
            s = i % 2
            sz = SIZES[i]
            if i + 1 < N_CHUNKS:
                loads[i + 1][0].wait()
                if i >= 1:
                    ydescs[i - 1].wait_send()
                nsz = SIZES[i + 1]
                own_bf16[(i + 1) % 2, :nsz] = own_f32[
                    (i + 1) % 2, :nsz
                ].astype(jnp.bfloat16)
                ydescs[i + 1] = y_rdma(i + 1)
                ydescs[i + 1].start()

            ydescs[i].wait_recv()

            loads[i][1].wait()
            y = (
                own_f32[s, :sz]
                + recv_y[i % 4, :sz].astype(jnp.float32)
                + resid_v[s, :sz]
            )
            ms = jnp.mean(y * y, axis=-1, keepdims=True)
            out_v[:sz] = y * lax.rsqrt(ms + EPS) * gamma_ref[...]

            if i >= 2:
                xdescs[i - 2].wait_send()
                stores[i - 2].wait()
                xstores[i - 2].wait()
            out_bf16[s, :sz] = out_v[:sz].astype(jnp.bfloat16)
            st = pltpu.make_async_copy(
                out_bf16.at[s, pl.ds(0, sz), :],
                out_ref.at[pl.ds(base_own + OFFS[i], sz), :],
                store_sems.at[s],
            )
            st.start()
            stores[i] = st
            xdescs[i] = x_rdma(i)
            xdescs[i].start()

            if i + 2 < N_CHUNKS:
                loads[i + 2] = start_loads(i + 2)

            if i >= 1:
                handle_xrecv(i - 1)

        handle_xrecv(N_CHUNKS - 1)
        for k in (N_CHUNKS - 2, N_CHUNKS - 1):
            stores[k].wait()
            xstores[k].wait()
            ydescs[k].wait_send()
            xdescs[k].wait_send()

    return pl.pallas_call(
        body,
        out_shape=jax.ShapeDtypeStruct((N_ROWS, N_COLS), jnp.bfloat16),
        in_specs=[
            pl.BlockSpec(memory_space=pltpu.MemorySpace.HBM),
            pl.BlockSpec(memory_space=pltpu.MemorySpace.HBM),
            pl.BlockSpec(memory_space=pltpu.VMEM),
        ],
        out_specs=pl.BlockSpec(memory_space=pltpu.MemorySpace.HBM),
        scratch_shapes=[
            pltpu.VMEM((2, MAXC, N_COLS), jnp.float32),
            pltpu.VMEM((2, MAXC, N_COLS), jnp.float32),
            pltpu.VMEM((2, MAXC, N_COLS), jnp.bfloat16),
            pltpu.VMEM((4, MAXC, N_COLS), jnp.bfloat16),
            pltpu.VMEM((MAXC, N_COLS), jnp.float32),
            pltpu.VMEM((2, MAXC, N_COLS), jnp.bfloat16),
            pltpu.VMEM((4, MAXC, N_COLS), jnp.bfloat16),
            pltpu.SemaphoreType.DMA((2,)),
            pltpu.SemaphoreType.DMA((2,)),
            pltpu.SemaphoreType.DMA((2,)),
            pltpu.SemaphoreType.DMA((2,)),
            pltpu.SemaphoreType.DMA((2,)),
            pltpu.SemaphoreType.DMA((4,)),
            pltpu.SemaphoreType.DMA((2,)),
            pltpu.SemaphoreType.DMA((4,)),
        ],
        compiler_params=pltpu.CompilerParams(
            collective_id=0,
            vmem_limit_bytes=60 * 1024 * 1024,
        ),
    )(partial, resid, gamma2)


# device time: 216452 ns/iter; 1.0941x vs baseline; 1.0149x over previous
import jax
import jax.numpy as jnp
from jax import lax
from jax.experimental import pallas as pl
from jax.experimental.pallas import tpu as pltpu

N_ROWS = 8192
N_COLS = 2048
HALF = N_ROWS // 2
CHUNK = 128
N_CHUNKS = HALF // CHUNK
EPS = 1e-6


def kernel(partial, resid, gamma):
    gamma2 = gamma.reshape(1, N_COLS)

    def body(
        partial_ref,
        resid_ref,
        gamma_ref,
        out_ref,
        own_f32,
        resid_v,
        own_bf16,
        recv_y,
        out_v,
        out_bf16,
        recv_x,
        pload_sems,
        rload_sems,
        store_sems,
        xstore_sems,
        ysend_sems,
        yrecv_sems,
        xsend_sems,
        xrecv_sems,
    ):
        my_x = lax.axis_index("x")
        my_y = lax.axis_index("y")
        ynbr = (my_x, 1 - my_y)
        xnbr = (1 - my_x, my_y)
        base_own = my_x * HALF
        base_oth = (1 - my_x) * HALF

        barrier_sem = pltpu.get_barrier_semaphore()
        for nbr in (ynbr, xnbr):
            pl.semaphore_signal(
                barrier_sem, inc=1, device_id=nbr,
                device_id_type=pl.DeviceIdType.MESH,
            )
        pl.semaphore_wait(barrier_sem, 2)

        def start_loads(i):
            s = i % 2
            rows = pl.ds(base_own + i * CHUNK, CHUNK)
            ld_p = pltpu.make_async_copy(
                partial_ref.at[0, rows, :], own_f32.at[s], pload_sems.at[s]
            )
            ld_r = pltpu.make_async_copy(
                resid_ref.at[rows, :], resid_v.at[s], rload_sems.at[s]
            )
            ld_p.start()
            ld_r.start()
            return ld_p, ld_r

        def y_rdma(i):
            return pltpu.make_async_remote_copy(
                src_ref=own_bf16.at[i % 2],
                dst_ref=recv_y.at[i % 4],
                send_sem=ysend_sems.at[i % 2],
                recv_sem=yrecv_sems.at[i % 4],
                device_id=ynbr,
                device_id_type=pl.DeviceIdType.MESH,
            )

        def x_rdma(i):
            return pltpu.make_async_remote_copy(
                src_ref=out_bf16.at[i % 2],
                dst_ref=recv_x.at[i % 4],
                send_sem=xsend_sems.at[i % 2],
                recv_sem=xrecv_sems.at[i % 4],
                device_id=xnbr,
                device_id_type=pl.DeviceIdType.MESH,
            )

        loads = [None] * N_CHUNKS
        ydescs = [None] * N_CHUNKS
        xdescs = [None] * N_CHUNKS
        stores = [None] * N_CHUNKS
        xstores = [None] * N_CHUNKS

        def handle_xrecv(j):
            xdescs[j].wait_recv()
            st = pltpu.make_async_copy(
                recv_x.at[j % 4],
                out_ref.at[pl.ds(base_oth + j * CHUNK, CHUNK), :],
                xstore_sems.at[j % 2],
            )
            st.start()
            xstores[j] = st

        loads[0] = start_loads(0)
        loads[1] = start_loads(1)
        loads[0][0].wait()
        own_bf16[0] = own_f32[0].astype(jnp.bfloat16)
        ydescs[0] = y_rdma(0)
        ydescs[0].start()

        for i in range(N_CHUNKS):
            s = i % 2
            if i + 1 < N_CHUNKS:
                loads[i + 1][0].wait()
                if i >= 1:
                    ydescs[i - 1].wait_send()
                own_bf16[(i + 1) % 2] = own_f32[(i + 1) % 2].astype(jnp.bfloat16)
                ydescs[i + 1] = y_rdma(i + 1)
                ydescs[i + 1].start()

            ydescs[i].wait_recv()

            loads[i][1].wait()
            y = own_f32[s] + recv_y[i % 4].astype(jnp.float32) + resid_v[s]
            ms = jnp.mean(y * y, axis=-1, keepdims=True)
            out_v[...] = y * lax.rsqrt(ms + EPS) * gamma_ref[...]

            if i >= 2:
                xdescs[i - 2].wait_send()
                stores[i - 2].wait()
                xstores[i - 2].wait()
            out_bf16[s] = out_v[...].astype(jnp.bfloat16)
            st = pltpu.make_async_copy(
                out_bf16.at[s],
                out_ref.at[pl.ds(base_own + i * CHUNK, CHUNK), :],
                store_sems.at[s],
            )
            st.start()
            stores[i] = st
            xdescs[i] = x_rdma(i)
            xdescs[i].start()

            if i + 2 < N_CHUNKS:
                loads[i + 2] = start_loads(i + 2)

            if i >= 1:
                handle_xrecv(i - 1)

        handle_xrecv(N_CHUNKS - 1)
        for k in (N_CHUNKS - 2, N_CHUNKS - 1):
            stores[k].wait()
            xstores[k].wait()
            ydescs[k].wait_send()
            xdescs[k].wait_send()

    return pl.pallas_call(
        body,
        out_shape=jax.ShapeDtypeStruct((N_ROWS, N_COLS), jnp.bfloat16),
        in_specs=[
            pl.BlockSpec(memory_space=pltpu.MemorySpace.HBM),
            pl.BlockSpec(memory_space=pltpu.MemorySpace.HBM),
            pl.BlockSpec(memory_space=pltpu.VMEM),
        ],
        out_specs=pl.BlockSpec(memory_space=pltpu.MemorySpace.HBM),
        scratch_shapes=[
            pltpu.VMEM((2, CHUNK, N_COLS), jnp.float32),
            pltpu.VMEM((2, CHUNK, N_COLS), jnp.float32),
            pltpu.VMEM((2, CHUNK, N_COLS), jnp.bfloat16),
            pltpu.VMEM((4, CHUNK, N_COLS), jnp.bfloat16),
            pltpu.VMEM((CHUNK, N_COLS), jnp.float32),
            pltpu.VMEM((2, CHUNK, N_COLS), jnp.bfloat16),
            pltpu.VMEM((4, CHUNK, N_COLS), jnp.bfloat16),
            pltpu.SemaphoreType.DMA((2,)),
            pltpu.SemaphoreType.DMA((2,)),
            pltpu.SemaphoreType.DMA((2,)),
            pltpu.SemaphoreType.DMA((2,)),
            pltpu.SemaphoreType.DMA((2,)),
            pltpu.SemaphoreType.DMA((4,)),
            pltpu.SemaphoreType.DMA((2,)),
            pltpu.SemaphoreType.DMA((4,)),
        ],
        compiler_params=pltpu.CompilerParams(
            collective_id=0,
            vmem_limit_bytes=60 * 1024 * 1024,
        ),
    )(partial, resid, gamma2)
